# baseline (speedup 1.0000x reference)
import jax
import jax.numpy as jnp
from jax import lax
from jax.experimental import pallas as pl
from jax.experimental.pallas import tpu as pltpu
from jax.experimental.pallas import tpu_sc as plsc

VOCAB = 1000000
BLOCK = 200
EMBED = 64
B = 4096
L = 200
R = B * L
C = 256
GPC = C // 128
CPL = B // C
ITEMS = R // C
PADW = 2 * EMBED

NC, NS = 2, 16
NW = NC * NS
IPT = ITEMS // NW
NBUF = 2


def _sc_body(emb_hbm, xt_hbm, pos_hbm, out_hbm, idx_v, rows_v, pos_v,
             gsem, osem):
    wid = lax.axis_index("s") * NC + lax.axis_index("c")
    base = wid * IPT
    pltpu.sync_copy(pos_hbm, pos_v)
    pltpu.sync_copy(xt_hbm.at[pl.ds(base * GPC, IPT * GPC)], idx_v)

    def fire(t, s):
        for h in range(GPC):
            pltpu.async_copy(
                emb_hbm.at[idx_v.at[t * GPC + h]],
                rows_v.at[s].at[pl.ds(h * 128, 128)], gsem[s])

    def gwait(t, s):
        for h in range(GPC):
            pltpu.make_async_copy(
                emb_hbm.at[idx_v.at[t * GPC + h]],
                rows_v.at[s].at[pl.ds(h * 128, 128)], gsem[s]).wait()

    def out_desc(t, s):
        i = base + t
        l = i // CPL
        b0 = (i % CPL) * C
        return pltpu.make_async_copy(
            rows_v.at[s], out_hbm.at[l, pl.ds(b0, C)], osem[s])

    def consume(t, s):
        i = base + t
        l = i // CPL
        gwait(t, s)
        p = [pos_v[l, pl.ds(k * 16, 16)] for k in range(EMBED // 16)]

        @plsc.parallel_loop(0, C, 1, unroll=4)
        def _(j):
            for k in range(EMBED // 16):
                rows_v[s, j, pl.ds(k * 16, 16)] += p[k]

        out_desc(t, s).start()

    for s in range(NBUF - 1):
        fire(s, s)

    @pl.loop(0, IPT, step=NBUF)
    def _(t):
        for s in range(NBUF):
            tt = t + s
            ns = (s + NBUF - 1) % NBUF

            @pl.when(tt + NBUF - 1 < IPT)
            def _():
                @pl.when(tt > 0)
                def _():
                    out_desc(tt - 1, ns).wait()
                fire(tt + NBUF - 1, ns)

            consume(tt, s)

    for s in range(NBUF):
        out_desc(IPT - NBUF + s, s).wait()


NCOL = 32768
NBLK = (VOCAB + NCOL - 1) // NCOL


def _tc_body(et_ref, out_ref):
    out_ref[:, :EMBED] = jnp.swapaxes(et_ref[...], 0, 1)


def _tc_pad_transpose(emb_t):
    return pl.pallas_call(
        _tc_body,
        grid=(NBLK,),
        in_specs=[pl.BlockSpec((EMBED, NCOL), lambda j: (0, j))],
        out_specs=pl.BlockSpec((NCOL, PADW), lambda j: (j, 0)),
        out_shape=jax.ShapeDtypeStruct((VOCAB, PADW), jnp.float32),
    )(emb_t)


@jax.jit
def _sc_lookup(embp, xt, pos_table):
    mesh = plsc.VectorSubcoreMesh(core_axis_name="c", subcore_axis_name="s")
    return pl.kernel(
        _sc_body,
        out_type=jax.ShapeDtypeStruct((L, B, PADW), jnp.float32),
        mesh=mesh,
        compiler_params=pltpu.CompilerParams(
            use_tc_tiling_on_sc=True, needs_layout_passes=False),
        scratch_types=[
            pltpu.VMEM((IPT * GPC, 128), jnp.int32),
            pltpu.VMEM((NBUF, C, PADW), jnp.float32),
            pltpu.VMEM((BLOCK, EMBED), jnp.float32),
            [pltpu.SemaphoreType.DMA] * NBUF,
            [pltpu.SemaphoreType.DMA] * NBUF,
        ],
    )(embp, xt, pos_table)


def kernel(x, emb_table, pos_table):
    xt = x.T.reshape(R // 128, 128)
    embp = _tc_pad_transpose(emb_table.T)
    out = _sc_lookup(embp, xt, pos_table)
    return out[:, :, :EMBED]


# speedup vs baseline: 1.0029x; 1.0029x over previous
import jax
import jax.numpy as jnp
from jax import lax
from jax.experimental import pallas as pl
from jax.experimental.pallas import tpu as pltpu
from jax.experimental.pallas import tpu_sc as plsc

VOCAB = 1000000
BLOCK = 200
EMBED = 64
B = 4096
L = 200
R = B * L
C = 128
CPL = B // C
ITEMS = R // C
PADW = 2 * EMBED

NC, NS = 2, 16
NW = NC * NS
IPT = ITEMS // NW
NBUF = 4


def _sc_body(emb_hbm, xt_hbm, pos_hbm, out_hbm, idx_v, rows_v, pos_v,
             gsem, osem):
    wid = lax.axis_index("s") * NC + lax.axis_index("c")
    base = wid * IPT
    pltpu.sync_copy(pos_hbm, pos_v)
    pltpu.sync_copy(xt_hbm.at[pl.ds(base, IPT)], idx_v)

    def fire(t, s):
        pltpu.async_copy(emb_hbm.at[idx_v.at[t]], rows_v.at[s], gsem[s])

    def gwait(t, s):
        pltpu.make_async_copy(
            emb_hbm.at[idx_v.at[t]], rows_v.at[s], gsem[s]).wait()

    def out_desc(t, s):
        i = base + t
        l = i // CPL
        b0 = (i % CPL) * C
        return pltpu.make_async_copy(
            rows_v.at[s], out_hbm.at[l, pl.ds(b0, C)], osem[s])

    def consume(t, s):
        i = base + t
        l = i // CPL
        gwait(t, s)
        p = [pos_v[l, pl.ds(k * 16, 16)] for k in range(EMBED // 16)]

        @plsc.parallel_loop(0, C, 1, unroll=4)
        def _(j):
            for k in range(EMBED // 16):
                rows_v[s, j, pl.ds(k * 16, 16)] += p[k]

        out_desc(t, s).start()

    for s in range(NBUF - 1):
        fire(s, s)

    @pl.loop(0, IPT, step=NBUF)
    def _(t):
        for s in range(NBUF):
            tt = t + s
            ns = (s + NBUF - 1) % NBUF

            @pl.when(tt + NBUF - 1 < IPT)
            def _():
                @pl.when(tt > 0)
                def _():
                    out_desc(tt - 1, ns).wait()
                fire(tt + NBUF - 1, ns)

            consume(tt, s)

    for s in range(NBUF):
        out_desc(IPT - NBUF + s, s).wait()


NCOL = 32768
NBLK = (VOCAB + NCOL - 1) // NCOL


def _tc_body(et_ref, out_ref):
    out_ref[:, :EMBED] = jnp.swapaxes(et_ref[...], 0, 1)


def _tc_pad_transpose(emb_t):
    return pl.pallas_call(
        _tc_body,
        grid=(NBLK,),
        in_specs=[pl.BlockSpec((EMBED, NCOL), lambda j: (0, j))],
        out_specs=pl.BlockSpec((NCOL, PADW), lambda j: (j, 0)),
        out_shape=jax.ShapeDtypeStruct((VOCAB, PADW), jnp.float32),
    )(emb_t)


@jax.jit
def _sc_lookup(embp, xt, pos_table):
    mesh = plsc.VectorSubcoreMesh(core_axis_name="c", subcore_axis_name="s")
    return pl.kernel(
        _sc_body,
        out_type=jax.ShapeDtypeStruct((L, B, PADW), jnp.float32),
        mesh=mesh,
        compiler_params=pltpu.CompilerParams(
            use_tc_tiling_on_sc=True, needs_layout_passes=False),
        scratch_types=[
            pltpu.VMEM((IPT, C), jnp.int32),
            pltpu.VMEM((NBUF, C, PADW), jnp.float32),
            pltpu.VMEM((BLOCK, EMBED), jnp.float32),
            [pltpu.SemaphoreType.DMA] * NBUF,
            [pltpu.SemaphoreType.DMA] * NBUF,
        ],
    )(embp, xt, pos_table)


def kernel(x, emb_table, pos_table):
    xt = x.T.reshape(R // C, C)
    embp = _tc_pad_transpose(emb_table.T)
    out = _sc_lookup(embp, xt, pos_table)
    return out[:, :, :EMBED]
